# baseline (speedup 1.0000x reference)
import jax
import jax.numpy as jnp
from jax.experimental import pallas as pl
from jax.experimental.pallas import tpu as pltpu

Q = 1024
D = 32
K = 100000
TOPK = 16
CHUNK = 1024
SEG = 16
L = CHUNK // SEG
KPAD = 102400
PAD_VAL = 1e17
IBIG = 2**31 - 1
UNROLL = 4


def _sweep2(d):
    sm = d[0:SEG]
    sm2 = jnp.full((SEG, Q), jnp.inf, dtype=jnp.float32)
    pos = jnp.zeros((SEG, Q), dtype=jnp.int32)
    for l in range(1, L):
        x = d[l * SEG:(l + 1) * SEG]
        isnew = x < sm
        sm2 = jnp.where(isnew, sm, jnp.minimum(sm2, x))
        pos = jnp.where(isnew, l, pos)
        sm = jnp.where(isnew, x, sm)
    return sm, pos, sm2


def _knn_body(q_ref, k_ref, x2_ref, y2_ref, ov_ref, oi_ref, bv_ref, bi_ref):
    j = pl.program_id(0)
    nchunks = pl.num_programs(0)

    @pl.when(j == 0)
    def _init():
        bv_ref[...] = jnp.full((TOPK, Q), jnp.inf, dtype=jnp.float32)
        bi_ref[...] = jnp.full((TOPK, Q), IBIG, dtype=jnp.int32)

    x = q_ref[...]
    kc = k_ref[...]
    x2 = x2_ref[...]
    y2 = y2_ref[...]
    inner = -2.0 * jax.lax.dot_general(
        kc, x, (((1,), (1,)), ((), ())),
        preferred_element_type=jnp.float32)
    d = (x2 + inner) + y2

    siota = jax.lax.broadcasted_iota(jnp.int32, (SEG, Q), 0)
    i16 = jax.lax.broadcasted_iota(jnp.int32, (TOPK, Q), 0)
    base = j * CHUNK

    def _consume(sm, gis, bv, bi):
        def inner_cond(ist):
            sm_i, cons_i, bv_i, bi_i = ist
            tv = bv_i[TOPK - 1:TOPK]
            ti = bi_i[TOPK - 1:TOPK]
            return jnp.any((sm_i < tv) | ((sm_i == tv) & (gis < ti)))

        def inner_body(ist):
            sm_i, cons_i, bv_i, bi_i = ist
            for _ in range(UNROLL):
                m_i = jnp.min(sm_i, axis=0, keepdims=True)
                g_i = jnp.min(jnp.where(sm_i == m_i, gis, IBIG), axis=0,
                              keepdims=True)
                less = (bv_i < m_i) | ((bv_i == m_i) & (bi_i < g_i))
                p = jnp.sum(less.astype(jnp.int32), axis=0, keepdims=True)
                bv_sh = jnp.concatenate([bv_i[:1], bv_i[:-1]], axis=0)
                bi_sh = jnp.concatenate([bi_i[:1], bi_i[:-1]], axis=0)
                bv_i = jnp.where(i16 < p, bv_i,
                                 jnp.where(i16 == p, m_i, bv_sh))
                bi_i = jnp.where(i16 < p, bi_i,
                                 jnp.where(i16 == p, g_i, bi_sh))
                seg = (g_i - base) & (SEG - 1)
                onehot = siota == seg
                cons_i = jnp.where(onehot, jnp.int32(1), cons_i)
                sm_i = jnp.where(onehot, jnp.inf, sm_i)
            return (sm_i, cons_i, bv_i, bi_i)

        cons0 = jnp.zeros((SEG, Q), dtype=jnp.int32)
        _, cons, bv, bi = jax.lax.while_loop(
            inner_cond, inner_body, (sm, cons0, bv, bi))
        return cons, bv, bi

    def _need_more(cons, sm2, bv):
        tv = bv[TOPK - 1:TOPK]
        return jnp.any((cons == 1) & (sm2 <= tv)).astype(jnp.int32)

    sm0, pos0, sm20 = _sweep2(d)
    bv = bv_ref[...]
    bi = bi_ref[...]
    cons, bv, bi = _consume(sm0, (base + siota) + pos0 * SEG, bv, bi)
    need = _need_more(cons, sm20, bv)

    def rare_cond(st):
        need_ = st[5]
        return need_ == 1

    def rare_body(st):
        d_, bv_, bi_, cons_, pos_, need_ = st
        d_ = jnp.concatenate(
            [jnp.where((cons_ == 1) & (pos_ == l), jnp.inf,
                       d_[l * SEG:(l + 1) * SEG])
             for l in range(L)], axis=0)
        sm_, pos_n, sm2_ = _sweep2(d_)
        cons_n, bv_, bi_ = _consume(sm_, (base + siota) + pos_n * SEG,
                                    bv_, bi_)
        need_n = _need_more(cons_n, sm2_, bv_)
        return (d_, bv_, bi_, cons_n, pos_n, need_n)

    _, bv, bi, _, _, _ = jax.lax.while_loop(
        rare_cond, rare_body, (d, bv, bi, cons, pos0, need))
    bv_ref[...] = bv
    bi_ref[...] = bi

    @pl.when(j == nchunks - 1)
    def _emit():
        ov_ref[...] = -bv
        oi_ref[...] = bi


def _knn_call(queries, keys_pad, x2, y2_pad):
    return pl.pallas_call(
        _knn_body,
        grid=(KPAD // CHUNK,),
        in_specs=[
            pl.BlockSpec((Q, D), lambda j: (0, 0)),
            pl.BlockSpec((CHUNK, D), lambda j: (j, 0)),
            pl.BlockSpec((1, Q), lambda j: (0, 0)),
            pl.BlockSpec((CHUNK, 1), lambda j: (j, 0)),
        ],
        out_specs=[
            pl.BlockSpec((TOPK, Q), lambda j: (0, 0)),
            pl.BlockSpec((TOPK, Q), lambda j: (0, 0)),
        ],
        out_shape=[
            jax.ShapeDtypeStruct((TOPK, Q), jnp.float32),
            jax.ShapeDtypeStruct((TOPK, Q), jnp.int32),
        ],
        scratch_shapes=[
            pltpu.VMEM((TOPK, Q), jnp.float32),
            pltpu.VMEM((TOPK, Q), jnp.int32),
        ],
    )(queries, keys_pad, x2, y2_pad)


def kernel(queries, keys, k):
    del k
    keys_pad = jnp.concatenate(
        [keys, jnp.full((KPAD - K, D), PAD_VAL, dtype=keys.dtype)], axis=0)
    x2 = jnp.sum(queries * queries, axis=-1, keepdims=True).T
    y2_pad = jnp.sum(keys_pad * keys_pad, axis=-1, keepdims=True)
    neg_vals_t, nn_idx_t = _knn_call(queries, keys_pad, x2, y2_pad)
    neg_vals = neg_vals_t.T
    nn_idx = nn_idx_t.T
    center_idx = jnp.tile(
        jnp.arange(Q, dtype=nn_idx.dtype)[:, None], (1, TOPK))
    edge_index = jnp.stack((nn_idx, center_idx), axis=0)
    return neg_vals, edge_index


# speedup vs baseline: 1.0229x; 1.0229x over previous
import jax
import jax.numpy as jnp
from jax.experimental import pallas as pl
from jax.experimental.pallas import tpu as pltpu

Q = 1024
D = 32
K = 100000
TOPK = 16
CHUNK = 1024
SEG = 16
L = CHUNK // SEG
KPAD = 102400
PAD_VAL = 1e17
IBIG = 2**31 - 1
UNROLL = 2


def _sweep2(d):
    sm = d[0:SEG]
    sm2 = jnp.full((SEG, Q), jnp.inf, dtype=jnp.float32)
    pos = jnp.zeros((SEG, Q), dtype=jnp.int32)
    for l in range(1, L):
        x = d[l * SEG:(l + 1) * SEG]
        isnew = x < sm
        sm2 = jnp.where(isnew, sm, jnp.minimum(sm2, x))
        pos = jnp.where(isnew, l, pos)
        sm = jnp.where(isnew, x, sm)
    return sm, pos, sm2


def _knn_body(q_ref, k_ref, x2_ref, y2_ref, ov_ref, oi_ref, bv_ref, bi_ref):
    j = pl.program_id(0)
    nchunks = pl.num_programs(0)

    @pl.when(j == 0)
    def _init():
        bv_ref[...] = jnp.full((TOPK, Q), jnp.inf, dtype=jnp.float32)
        bi_ref[...] = jnp.full((TOPK, Q), IBIG, dtype=jnp.int32)

    x = q_ref[...]
    kc = k_ref[...]
    x2 = x2_ref[...]
    y2 = y2_ref[...]
    inner = -2.0 * jax.lax.dot_general(
        kc, x, (((1,), (1,)), ((), ())),
        preferred_element_type=jnp.float32)
    d = (x2 + inner) + y2

    siota = jax.lax.broadcasted_iota(jnp.int32, (SEG, Q), 0)
    i16 = jax.lax.broadcasted_iota(jnp.int32, (TOPK, Q), 0)
    base = j * CHUNK

    def _consume(sm, gis, bv, bi):
        def inner_cond(ist):
            sm_i, cons_i, bv_i, bi_i = ist
            tv = bv_i[TOPK - 1:TOPK]
            return jnp.any(sm_i <= tv)

        def inner_body(ist):
            sm_i, cons_i, bv_i, bi_i = ist
            for _ in range(UNROLL):
                m_i = jnp.min(sm_i, axis=0, keepdims=True)
                g_i = jnp.min(jnp.where(sm_i == m_i, gis, IBIG), axis=0,
                              keepdims=True)
                less = (bv_i < m_i) | ((bv_i == m_i) & (bi_i < g_i))
                p = jnp.sum(less.astype(jnp.int32), axis=0, keepdims=True)
                bv_sh = jnp.concatenate([bv_i[:1], bv_i[:-1]], axis=0)
                bi_sh = jnp.concatenate([bi_i[:1], bi_i[:-1]], axis=0)
                bv_i = jnp.where(i16 < p, bv_i,
                                 jnp.where(i16 == p, m_i, bv_sh))
                bi_i = jnp.where(i16 < p, bi_i,
                                 jnp.where(i16 == p, g_i, bi_sh))
                seg = (g_i - base) & (SEG - 1)
                onehot = siota == seg
                cons_i = jnp.where(onehot, jnp.int32(1), cons_i)
                sm_i = jnp.where(onehot, jnp.inf, sm_i)
            return (sm_i, cons_i, bv_i, bi_i)

        cons0 = jnp.zeros((SEG, Q), dtype=jnp.int32)
        _, cons, bv, bi = jax.lax.while_loop(
            inner_cond, inner_body, (sm, cons0, bv, bi))
        return cons, bv, bi

    def _need_more(cons, sm2, bv):
        tv = bv[TOPK - 1:TOPK]
        return jnp.any((cons == 1) & (sm2 <= tv)).astype(jnp.int32)

    sm0, pos0, sm20 = _sweep2(d)
    bv = bv_ref[...]
    bi = bi_ref[...]
    cons, bv, bi = _consume(sm0, (base + siota) + pos0 * SEG, bv, bi)
    need = _need_more(cons, sm20, bv)

    def rare_cond(st):
        need_ = st[5]
        return need_ == 1

    def rare_body(st):
        d_, bv_, bi_, cons_, pos_, need_ = st
        d_ = jnp.concatenate(
            [jnp.where((cons_ == 1) & (pos_ == l), jnp.inf,
                       d_[l * SEG:(l + 1) * SEG])
             for l in range(L)], axis=0)
        sm_, pos_n, sm2_ = _sweep2(d_)
        cons_n, bv_, bi_ = _consume(sm_, (base + siota) + pos_n * SEG,
                                    bv_, bi_)
        need_n = _need_more(cons_n, sm2_, bv_)
        return (d_, bv_, bi_, cons_n, pos_n, need_n)

    _, bv, bi, _, _, _ = jax.lax.while_loop(
        rare_cond, rare_body, (d, bv, bi, cons, pos0, need))
    bv_ref[...] = bv
    bi_ref[...] = bi

    @pl.when(j == nchunks - 1)
    def _emit():
        ov_ref[...] = -bv
        oi_ref[...] = bi


def _knn_call(queries, keys_pad, x2, y2_pad):
    return pl.pallas_call(
        _knn_body,
        grid=(KPAD // CHUNK,),
        in_specs=[
            pl.BlockSpec((Q, D), lambda j: (0, 0)),
            pl.BlockSpec((CHUNK, D), lambda j: (j, 0)),
            pl.BlockSpec((1, Q), lambda j: (0, 0)),
            pl.BlockSpec((CHUNK, 1), lambda j: (j, 0)),
        ],
        out_specs=[
            pl.BlockSpec((TOPK, Q), lambda j: (0, 0)),
            pl.BlockSpec((TOPK, Q), lambda j: (0, 0)),
        ],
        out_shape=[
            jax.ShapeDtypeStruct((TOPK, Q), jnp.float32),
            jax.ShapeDtypeStruct((TOPK, Q), jnp.int32),
        ],
        scratch_shapes=[
            pltpu.VMEM((TOPK, Q), jnp.float32),
            pltpu.VMEM((TOPK, Q), jnp.int32),
        ],
    )(queries, keys_pad, x2, y2_pad)


def kernel(queries, keys, k):
    del k
    keys_pad = jnp.concatenate(
        [keys, jnp.full((KPAD - K, D), PAD_VAL, dtype=keys.dtype)], axis=0)
    x2 = jnp.sum(queries * queries, axis=-1, keepdims=True).T
    y2_pad = jnp.sum(keys_pad * keys_pad, axis=-1, keepdims=True)
    neg_vals_t, nn_idx_t = _knn_call(queries, keys_pad, x2, y2_pad)
    neg_vals = neg_vals_t.T
    nn_idx = nn_idx_t.T
    center_idx = jnp.tile(
        jnp.arange(Q, dtype=nn_idx.dtype)[:, None], (1, TOPK))
    edge_index = jnp.stack((nn_idx, center_idx), axis=0)
    return neg_vals, edge_index


# speedup vs baseline: 1.1885x; 1.1619x over previous
import jax
import jax.numpy as jnp
from jax.experimental import pallas as pl
from jax.experimental.pallas import tpu as pltpu

Q = 1024
D = 32
K = 100000
TOPK = 16
CHUNK = 1024
SEG = 16
L = CHUNK // SEG
KPAD = 102400
PAD_VAL = 1e17
IBIG = 2**31 - 1
UNROLL = 2


def _sweep2(get):
    sm = get(0)
    sm2 = jnp.full((SEG, Q), jnp.inf, dtype=jnp.float32)
    pos = jnp.zeros((SEG, Q), dtype=jnp.int32)
    for l in range(1, L):
        x = get(l)
        isnew = x < sm
        sm2 = jnp.where(isnew, sm, jnp.minimum(sm2, x))
        pos = jnp.where(isnew, l, pos)
        sm = jnp.where(isnew, x, sm)
    return sm, pos, sm2


def _knn_body(q_ref, k_ref, x2_ref, y2_ref, ov_ref, oi_ref, bv_ref, bi_ref):
    j = pl.program_id(0)
    nchunks = pl.num_programs(0)

    @pl.when(j == 0)
    def _init():
        bv_ref[...] = jnp.full((TOPK, Q), jnp.inf, dtype=jnp.float32)
        bi_ref[...] = jnp.full((TOPK, Q), IBIG, dtype=jnp.int32)

    x = q_ref[...]
    kc = k_ref[...]
    x2 = x2_ref[...]
    y2 = y2_ref[...]
    inner = -2.0 * jax.lax.dot_general(
        kc, x, (((1,), (1,)), ((), ())),
        preferred_element_type=jnp.float32)

    def _dslice(l):
        return (x2 + inner[l * SEG:(l + 1) * SEG]) + y2[l * SEG:(l + 1) * SEG]

    siota = jax.lax.broadcasted_iota(jnp.int32, (SEG, Q), 0)
    i16 = jax.lax.broadcasted_iota(jnp.int32, (TOPK, Q), 0)
    base = j * CHUNK

    def _consume(sm, gis, bv, bi):
        def inner_cond(ist):
            sm_i, cons_i, bv_i, bi_i = ist
            tv = bv_i[TOPK - 1:TOPK]
            return jnp.any(sm_i <= tv)

        def inner_body(ist):
            sm_i, cons_i, bv_i, bi_i = ist
            for _ in range(UNROLL):
                m_i = jnp.min(sm_i, axis=0, keepdims=True)
                g_i = jnp.min(jnp.where(sm_i == m_i, gis, IBIG), axis=0,
                              keepdims=True)
                less = (bv_i < m_i) | ((bv_i == m_i) & (bi_i < g_i))
                p = jnp.sum(less.astype(jnp.int32), axis=0, keepdims=True)
                bv_sh = jnp.concatenate([bv_i[:1], bv_i[:-1]], axis=0)
                bi_sh = jnp.concatenate([bi_i[:1], bi_i[:-1]], axis=0)
                bv_i = jnp.where(i16 < p, bv_i,
                                 jnp.where(i16 == p, m_i, bv_sh))
                bi_i = jnp.where(i16 < p, bi_i,
                                 jnp.where(i16 == p, g_i, bi_sh))
                seg = (g_i - base) & (SEG - 1)
                onehot = siota == seg
                cons_i = jnp.where(onehot, jnp.int32(1), cons_i)
                sm_i = jnp.where(onehot, jnp.inf, sm_i)
            return (sm_i, cons_i, bv_i, bi_i)

        cons0 = jnp.zeros((SEG, Q), dtype=jnp.int32)
        _, cons, bv, bi = jax.lax.while_loop(
            inner_cond, inner_body, (sm, cons0, bv, bi))
        return cons, bv, bi

    def _need_more(cons, sm2, bv):
        tv = bv[TOPK - 1:TOPK]
        return jnp.any((cons == 1) & (sm2 <= tv)).astype(jnp.int32)

    sm0, pos0, sm20 = _sweep2(_dslice)
    bv = bv_ref[...]
    bi = bi_ref[...]
    cons, bv, bi = _consume(sm0, (base + siota) + pos0 * SEG, bv, bi)
    need = _need_more(cons, sm20, bv)

    def rare_cond(st):
        need_ = st[6]
        return need_ == 1

    def rare_body(st):
        bv_, bi_, cons_, pos_, clo_, chi_, need_ = st
        hit = cons_ == 1
        clo_ = jnp.where(hit & (pos_ < 32),
                         clo_ | jnp.left_shift(jnp.int32(1), pos_), clo_)
        chi_ = jnp.where(hit & (pos_ >= 32),
                         chi_ | jnp.left_shift(jnp.int32(1), pos_ - 32), chi_)

        def get(l):
            bits = clo_ if l < 32 else chi_
            eaten = ((bits >> (l % 32)) & 1) == 1
            return jnp.where(eaten, jnp.inf, _dslice(l))

        sm_, pos_n, sm2_ = _sweep2(get)
        cons_n, bv_, bi_ = _consume(sm_, (base + siota) + pos_n * SEG,
                                    bv_, bi_)
        need_n = _need_more(cons_n, sm2_, bv_)
        return (bv_, bi_, cons_n, pos_n, clo_, chi_, need_n)

    zero16 = jnp.zeros((SEG, Q), dtype=jnp.int32)
    bv, bi, _, _, _, _, _ = jax.lax.while_loop(
        rare_cond, rare_body, (bv, bi, cons, pos0, zero16, zero16, need))
    bv_ref[...] = bv
    bi_ref[...] = bi

    @pl.when(j == nchunks - 1)
    def _emit():
        ov_ref[...] = -bv
        oi_ref[...] = bi


def _knn_call(queries, keys_pad, x2, y2_pad):
    return pl.pallas_call(
        _knn_body,
        grid=(KPAD // CHUNK,),
        in_specs=[
            pl.BlockSpec((Q, D), lambda j: (0, 0)),
            pl.BlockSpec((CHUNK, D), lambda j: (j, 0)),
            pl.BlockSpec((1, Q), lambda j: (0, 0)),
            pl.BlockSpec((CHUNK, 1), lambda j: (j, 0)),
        ],
        out_specs=[
            pl.BlockSpec((TOPK, Q), lambda j: (0, 0)),
            pl.BlockSpec((TOPK, Q), lambda j: (0, 0)),
        ],
        out_shape=[
            jax.ShapeDtypeStruct((TOPK, Q), jnp.float32),
            jax.ShapeDtypeStruct((TOPK, Q), jnp.int32),
        ],
        scratch_shapes=[
            pltpu.VMEM((TOPK, Q), jnp.float32),
            pltpu.VMEM((TOPK, Q), jnp.int32),
        ],
    )(queries, keys_pad, x2, y2_pad)


def kernel(queries, keys, k):
    del k
    keys_pad = jnp.concatenate(
        [keys, jnp.full((KPAD - K, D), PAD_VAL, dtype=keys.dtype)], axis=0)
    x2 = jnp.sum(queries * queries, axis=-1, keepdims=True).T
    y2_pad = jnp.sum(keys_pad * keys_pad, axis=-1, keepdims=True)
    neg_vals_t, nn_idx_t = _knn_call(queries, keys_pad, x2, y2_pad)
    neg_vals = neg_vals_t.T
    nn_idx = nn_idx_t.T
    center_idx = jnp.tile(
        jnp.arange(Q, dtype=nn_idx.dtype)[:, None], (1, TOPK))
    edge_index = jnp.stack((nn_idx, center_idx), axis=0)
    return neg_vals, edge_index
